# Initial kernel scaffold; baseline (speedup 1.0000x reference)
import jax
import jax.numpy as jnp
from jax.experimental import pallas as pl


def kernel(x, adj, W1, W2, We1, be1, We2, be2, We3, be3):
    raise NotImplementedError("write your pallas kernel here")


# speedup vs baseline: 1.0204x; 1.0204x over previous
import functools

import jax
import jax.numpy as jnp
from jax import lax
from jax.experimental import pallas as pl
from jax.experimental.pallas import tpu as pltpu, tpu_sc as plsc

_E = 160000
_CH = 128
_NC = 2
_NS = 16
_NW = _NC * _NS
_W = 128
_MLP_B = 1280


def _tc_matmul(x, w):
    def body(x_ref, w_ref, o_ref):
        o_ref[...] = jnp.dot(x_ref[...], w_ref[...],
                             preferred_element_type=jnp.float32,
                     precision=jax.lax.Precision.HIGHEST)
    return pl.pallas_call(
        body,
        out_shape=jax.ShapeDtypeStruct((x.shape[0], w.shape[1]), jnp.float32),
    )(x, w)


def _tc_combine_relu_matmul(parts, w_pad):
    def body(p_ref, w_ref, o_ref):
        a = jnp.maximum(p_ref[0] + p_ref[1], 0.0)
        o_ref[...] = jnp.dot(a, w_ref[...], preferred_element_type=jnp.float32,
                     precision=jax.lax.Precision.HIGHEST)
    return pl.pallas_call(
        body,
        out_shape=jax.ShapeDtypeStruct((parts.shape[1], w_pad.shape[1]),
                                       jnp.float32),
    )(parts, w_pad)


def _tc_combine(parts, C):
    def body(p_ref, o_ref):
        o_ref[...] = (p_ref[0] + p_ref[1])[:, :C]
    return pl.pallas_call(
        body,
        out_shape=jax.ShapeDtypeStruct((parts.shape[1], C), jnp.float32),
    )(parts)


def _tc_edge_mlp(f1, f2, vmask3, We1a, We1b, be1, We2, be2, We3p, be3r):
    E = f1.shape[0]
    B = _MLP_B
    grid = E // B

    def body(f1_ref, f2_ref, vm_ref, wa_ref, wb_ref, b1_ref, w2_ref, b2_ref,
             w3_ref, b3_ref, o_ref):
        h = jnp.dot(f1_ref[...], wa_ref[...],
                    preferred_element_type=jnp.float32,
                     precision=jax.lax.Precision.HIGHEST)
        h = h + jnp.dot(f2_ref[...], wb_ref[...],
                        preferred_element_type=jnp.float32,
                     precision=jax.lax.Precision.HIGHEST)
        h = jnp.maximum(h + b1_ref[...], 0.0)
        h = jnp.maximum(jnp.dot(h, w2_ref[...],
                                preferred_element_type=jnp.float32,
                     precision=jax.lax.Precision.HIGHEST)
                        + b2_ref[...], 0.0)
        w = jnp.dot(h, w3_ref[...], preferred_element_type=jnp.float32,
                     precision=jax.lax.Precision.HIGHEST)
        w = w + b3_ref[...]
        v = jnp.maximum(w[:, 0:1], 0.0)
        vm = jnp.reshape(vm_ref[...], (B, 1))
        o_ref[...] = jnp.broadcast_to(v * vm, (B, _W))

    return pl.pallas_call(
        body,
        grid=(grid,),
        in_specs=[
            pl.BlockSpec((B, _W), lambda i: (i, 0)),
            pl.BlockSpec((B, _W), lambda i: (i, 0)),
            pl.BlockSpec((1, 1, B), lambda i: (i, 0, 0)),
            pl.BlockSpec(We1a.shape, lambda i: (0, 0)),
            pl.BlockSpec(We1b.shape, lambda i: (0, 0)),
            pl.BlockSpec(be1.shape, lambda i: (0, 0)),
            pl.BlockSpec(We2.shape, lambda i: (0, 0)),
            pl.BlockSpec(be2.shape, lambda i: (0, 0)),
            pl.BlockSpec(We3p.shape, lambda i: (0, 0)),
            pl.BlockSpec((1, 128), lambda i: (0, 0)),
        ],
        out_specs=pl.BlockSpec((B, _W), lambda i: (i, 0)),
        out_shape=jax.ShapeDtypeStruct((E, _W), jnp.float32),
    )(f1, f2, vmask3, We1a, We1b, be1, We2, be2, We3p, be3r)


_MESH = plsc.VectorSubcoreMesh(core_axis_name="c", subcore_axis_name="s")


def _sc_spmm(tbl, r_eff, c_idx, vb, zeros, N):
    NP = N + 8
    E = r_eff.shape[0]
    TCH = E // _CH
    NT = (TCH + _NW - 1) // _NW
    scaled = vb is not None

    scratch = [
        pltpu.VMEM((_CH,), jnp.int32),
        pltpu.VMEM((_CH,), jnp.int32),
        pltpu.VMEM((_CH, _W), jnp.float32),
        pltpu.VMEM((_CH, _W), jnp.float32),
        pltpu.VMEM_SHARED((NP, _W), jnp.float32),
        pltpu.SemaphoreType.DMA,
    ]

    def body(*refs):
        if scaled:
            (tbl_ref, r_ref, c_ref, vb_ref, z_ref, out_ref,
             cidx_v, ridx_v, rows_v, vb_v, acc, sem) = refs
        else:
            (tbl_ref, r_ref, c_ref, z_ref, out_ref,
             cidx_v, ridx_v, rows_v, vb_v, acc, sem) = refs
        cid = lax.axis_index("c")
        sid = lax.axis_index("s")
        wid = sid * _NC + cid

        @pl.when(sid == 0)
        def _():
            pltpu.sync_copy(z_ref, acc)
        plsc.subcore_barrier()

        def step(t, carry):
            ch = t * _NW + wid

            @pl.when(ch < TCH)
            def _():
                base = ch * _CH
                pltpu.sync_copy(c_ref.at[pl.ds(base, _CH)], cidx_v)
                pltpu.async_copy(tbl_ref.at[cidx_v], rows_v, sem).wait()
                if scaled:
                    pltpu.sync_copy(vb_ref.at[pl.ds(base, _CH)], vb_v)

                    def mul(i, c2):
                        for off in range(0, _W, 16):
                            rows_v[i, pl.ds(off, 16)] = (
                                rows_v[i, pl.ds(off, 16)]
                                * vb_v[i, pl.ds(off, 16)])
                        return c2
                    lax.fori_loop(0, _CH, mul, 0)
                pltpu.sync_copy(r_ref.at[pl.ds(base, _CH)], ridx_v)
                pltpu.sync_copy(rows_v, acc.at[ridx_v], add=True)
            return carry

        lax.fori_loop(0, NT, step, 0)
        plsc.subcore_barrier()

        @pl.when(sid == 0)
        def _():
            pltpu.sync_copy(acc.at[pl.ds(0, N)], out_ref.at[cid])

    fn = functools.partial(
        pl.kernel, body,
        out_type=jax.ShapeDtypeStruct((2, N, _W), jnp.float32),
        mesh=_MESH,
        scratch_types=scratch,
    )
    if scaled:
        return fn()(tbl, r_eff, c_idx, vb, zeros)
    return fn()(tbl, r_eff, c_idx, zeros)


def _sc_gather2(emb, r_idx, c_idx):
    E = r_idx.shape[0]
    TCH = E // _CH
    NT = (TCH + _NW - 1) // _NW
    N = emb.shape[0]

    def body(emb_ref, r_ref, c_ref, f1_ref, f2_ref, idx_v, rows_v, tbl_sh,
             sem):
        cid = lax.axis_index("c")
        sid = lax.axis_index("s")
        wid = sid * _NC + cid

        @pl.when(sid == 0)
        def _():
            pltpu.sync_copy(emb_ref, tbl_sh)
        plsc.subcore_barrier()

        def step(t, carry):
            ch = t * _NW + wid

            @pl.when(ch < TCH)
            def _():
                base = ch * _CH
                pltpu.sync_copy(r_ref.at[pl.ds(base, _CH)], idx_v)
                pltpu.async_copy(tbl_sh.at[idx_v], rows_v, sem).wait()
                pltpu.sync_copy(rows_v, f1_ref.at[pl.ds(base, _CH)])
                pltpu.sync_copy(c_ref.at[pl.ds(base, _CH)], idx_v)
                pltpu.async_copy(tbl_sh.at[idx_v], rows_v, sem).wait()
                pltpu.sync_copy(rows_v, f2_ref.at[pl.ds(base, _CH)])
            return carry

        lax.fori_loop(0, NT, step, 0)

    return pl.kernel(
        body,
        out_type=(jax.ShapeDtypeStruct((E, _W), jnp.float32),
                  jax.ShapeDtypeStruct((E, _W), jnp.float32)),
        mesh=_MESH,
        scratch_types=[
            pltpu.VMEM((_CH,), jnp.int32),
            pltpu.VMEM((_CH, _W), jnp.float32),
            pltpu.VMEM_SHARED((N, _W), jnp.float32),
            pltpu.SemaphoreType.DMA,
        ],
    )(emb, r_idx, c_idx)


def kernel(x, adj, W1, W2, We1, be1, We2, be2, We3, be3):
    N, D = x.shape
    C = W1.shape[1]

    r, c = jnp.nonzero(adj, size=_E, fill_value=0)
    r = r.astype(jnp.int32)
    c = c.astype(jnp.int32)
    key = r * N + c
    valid = jnp.concatenate(
        [jnp.ones((1,), jnp.bool_), key[1:] > key[:-1]])
    vmask3 = valid.astype(jnp.float32).reshape(_E // _MLP_B, 1, _MLP_B)
    r_eff = jnp.where(valid, r, N)

    zeros = jnp.zeros((N + 8, _W), jnp.float32)

    We1a = jnp.pad(We1[:C], ((0, _W - C), (0, 0)))
    We1b = jnp.pad(We1[C:], ((0, _W - C), (0, 0)))
    be1r = be1.reshape(1, -1)
    be2r = be2.reshape(1, -1)
    We3p = jnp.pad(We3, ((0, 0), (0, 127)))
    be3r = jnp.broadcast_to(be3.reshape(1, 1), (1, 128))
    W2p = jnp.pad(W2, ((0, _W - C), (0, 0)))

    def lane_pad(a):
        return jnp.pad(a, ((0, 8), (0, _W - C)))

    xW1 = _tc_matmul(x, W1)
    xW1p = lane_pad(xW1)
    a1p = _sc_spmm(xW1p, r_eff, c, None, zeros, N)
    q1 = _tc_combine_relu_matmul(a1p, W2p)
    embp = _sc_spmm(lane_pad(q1), r_eff, c, None, zeros, N)
    emb = _tc_combine(embp, _W)

    f1, f2 = _sc_gather2(emb, r, c)
    vb = _tc_edge_mlp(f1, f2, vmask3, We1a, We1b, be1r, We2, be2r,
                      We3p, be3r)

    a2p = _sc_spmm(xW1p, r_eff, c, vb, zeros, N)
    q2 = _tc_combine_relu_matmul(a2p, W2p)
    zp = _sc_spmm(lane_pad(q2), r_eff, c, vb, zeros, N)
    return _tc_combine(zp, C)


# speedup vs baseline: 1.0355x; 1.0147x over previous
import functools

import jax
import jax.numpy as jnp
from jax import lax
from jax.experimental import pallas as pl
from jax.experimental.pallas import tpu as pltpu, tpu_sc as plsc

_E = 160000
_CH = 128
_NC = 2
_NS = 16
_NW = _NC * _NS
_W = 128
_MLP_B = 1280


def _tc_matmul(x, w):
    def body(x_ref, w_ref, o_ref):
        o_ref[...] = jnp.dot(x_ref[...], w_ref[...],
                             preferred_element_type=jnp.float32)
    return pl.pallas_call(
        body,
        out_shape=jax.ShapeDtypeStruct((x.shape[0], w.shape[1]), jnp.float32),
    )(x, w)


def _tc_combine_relu_matmul(parts, w_pad):
    def body(p_ref, w_ref, o_ref):
        a = jnp.maximum(p_ref[0] + p_ref[1], 0.0)
        o_ref[...] = jnp.dot(a, w_ref[...], preferred_element_type=jnp.float32)
    return pl.pallas_call(
        body,
        out_shape=jax.ShapeDtypeStruct((parts.shape[1], w_pad.shape[1]),
                                       jnp.float32),
    )(parts, w_pad)


def _tc_combine(parts, C):
    def body(p_ref, o_ref):
        o_ref[...] = (p_ref[0] + p_ref[1])[:, :C]
    return pl.pallas_call(
        body,
        out_shape=jax.ShapeDtypeStruct((parts.shape[1], C), jnp.float32),
    )(parts)


def _tc_edge_mlp(f1, f2, vmask3, We1a, We1b, be1, We2, be2, We3p, be3r):
    E = f1.shape[0]
    B = _MLP_B
    grid = E // B

    def body(f1_ref, f2_ref, vm_ref, wa_ref, wb_ref, b1_ref, w2_ref, b2_ref,
             w3_ref, b3_ref, o_ref):
        h = jnp.dot(f1_ref[...], wa_ref[...],
                    preferred_element_type=jnp.float32)
        h = h + jnp.dot(f2_ref[...], wb_ref[...],
                        preferred_element_type=jnp.float32)
        h = jnp.maximum(h + b1_ref[...], 0.0)
        h = jnp.maximum(jnp.dot(h, w2_ref[...],
                                preferred_element_type=jnp.float32)
                        + b2_ref[...], 0.0)
        w = jnp.dot(h, w3_ref[...], preferred_element_type=jnp.float32)
        w = w + b3_ref[...]
        v = jnp.maximum(w[:, 0:1], 0.0)
        vm = jnp.reshape(vm_ref[...], (B, 1))
        o_ref[...] = jnp.broadcast_to(v * vm, (B, _W))

    return pl.pallas_call(
        body,
        grid=(grid,),
        in_specs=[
            pl.BlockSpec((B, _W), lambda i: (i, 0)),
            pl.BlockSpec((B, _W), lambda i: (i, 0)),
            pl.BlockSpec((1, 1, B), lambda i: (i, 0, 0)),
            pl.BlockSpec(We1a.shape, lambda i: (0, 0)),
            pl.BlockSpec(We1b.shape, lambda i: (0, 0)),
            pl.BlockSpec(be1.shape, lambda i: (0, 0)),
            pl.BlockSpec(We2.shape, lambda i: (0, 0)),
            pl.BlockSpec(be2.shape, lambda i: (0, 0)),
            pl.BlockSpec(We3p.shape, lambda i: (0, 0)),
            pl.BlockSpec((1, 128), lambda i: (0, 0)),
        ],
        out_specs=pl.BlockSpec((B, _W), lambda i: (i, 0)),
        out_shape=jax.ShapeDtypeStruct((E, _W), jnp.float32),
    )(f1, f2, vmask3, We1a, We1b, be1, We2, be2, We3p, be3r)


_MESH = plsc.VectorSubcoreMesh(core_axis_name="c", subcore_axis_name="s")


def _sc_spmm(tbl, r_eff, c_idx, vb, zeros, N):
    NP = N + 8
    E = r_eff.shape[0]
    TCH = E // _CH
    NT = (TCH + _NW - 1) // _NW
    scaled = vb is not None

    scratch = [
        pltpu.VMEM((_CH,), jnp.int32),
        pltpu.VMEM((_CH,), jnp.int32),
        pltpu.VMEM((_CH, _W), jnp.float32),
        pltpu.VMEM((_CH, _W), jnp.float32),
        pltpu.VMEM_SHARED((NP, _W), jnp.float32),
        pltpu.SemaphoreType.DMA,
    ]

    def body(*refs):
        if scaled:
            (tbl_ref, r_ref, c_ref, vb_ref, z_ref, out_ref,
             cidx_v, ridx_v, rows_v, vb_v, acc, sem) = refs
        else:
            (tbl_ref, r_ref, c_ref, z_ref, out_ref,
             cidx_v, ridx_v, rows_v, vb_v, acc, sem) = refs
        cid = lax.axis_index("c")
        sid = lax.axis_index("s")
        wid = sid * _NC + cid

        @pl.when(sid == 0)
        def _():
            pltpu.sync_copy(z_ref, acc)
        plsc.subcore_barrier()

        def step(t, carry):
            ch = t * _NW + wid

            @pl.when(ch < TCH)
            def _():
                base = ch * _CH
                pltpu.sync_copy(c_ref.at[pl.ds(base, _CH)], cidx_v)
                pltpu.async_copy(tbl_ref.at[cidx_v], rows_v, sem).wait()
                if scaled:
                    pltpu.sync_copy(vb_ref.at[pl.ds(base, _CH)], vb_v)

                    def mul(i, c2):
                        for off in range(0, _W, 16):
                            rows_v[i, pl.ds(off, 16)] = (
                                rows_v[i, pl.ds(off, 16)]
                                * vb_v[i, pl.ds(off, 16)])
                        return c2
                    lax.fori_loop(0, _CH, mul, 0)
                pltpu.sync_copy(r_ref.at[pl.ds(base, _CH)], ridx_v)
                pltpu.sync_copy(rows_v, acc.at[ridx_v], add=True)
            return carry

        lax.fori_loop(0, NT, step, 0)
        plsc.subcore_barrier()

        @pl.when(sid == 0)
        def _():
            pltpu.sync_copy(acc.at[pl.ds(0, N)], out_ref.at[cid])

    fn = functools.partial(
        pl.kernel, body,
        out_type=jax.ShapeDtypeStruct((2, N, _W), jnp.float32),
        mesh=_MESH,
        scratch_types=scratch,
    )
    if scaled:
        return fn()(tbl, r_eff, c_idx, vb, zeros)
    return fn()(tbl, r_eff, c_idx, zeros)


def _sc_gather2(emb, r_idx, c_idx):
    E = r_idx.shape[0]
    TCH = E // _CH
    NT = (TCH + _NW - 1) // _NW
    N = emb.shape[0]

    def body(emb_ref, r_ref, c_ref, f1_ref, f2_ref, idx_v, rows_v, tbl_sh,
             sem):
        cid = lax.axis_index("c")
        sid = lax.axis_index("s")
        wid = sid * _NC + cid

        @pl.when(sid == 0)
        def _():
            pltpu.sync_copy(emb_ref, tbl_sh)
        plsc.subcore_barrier()

        def step(t, carry):
            ch = t * _NW + wid

            @pl.when(ch < TCH)
            def _():
                base = ch * _CH
                pltpu.sync_copy(r_ref.at[pl.ds(base, _CH)], idx_v)
                pltpu.async_copy(tbl_sh.at[idx_v], rows_v, sem).wait()
                pltpu.sync_copy(rows_v, f1_ref.at[pl.ds(base, _CH)])
                pltpu.sync_copy(c_ref.at[pl.ds(base, _CH)], idx_v)
                pltpu.async_copy(tbl_sh.at[idx_v], rows_v, sem).wait()
                pltpu.sync_copy(rows_v, f2_ref.at[pl.ds(base, _CH)])
            return carry

        lax.fori_loop(0, NT, step, 0)

    return pl.kernel(
        body,
        out_type=(jax.ShapeDtypeStruct((E, _W), jnp.float32),
                  jax.ShapeDtypeStruct((E, _W), jnp.float32)),
        mesh=_MESH,
        scratch_types=[
            pltpu.VMEM((_CH,), jnp.int32),
            pltpu.VMEM((_CH, _W), jnp.float32),
            pltpu.VMEM_SHARED((N, _W), jnp.float32),
            pltpu.SemaphoreType.DMA,
        ],
    )(emb, r_idx, c_idx)


def kernel(x, adj, W1, W2, We1, be1, We2, be2, We3, be3):
    N, D = x.shape
    C = W1.shape[1]

    r, c = jnp.nonzero(adj, size=_E, fill_value=0)
    r = r.astype(jnp.int32)
    c = c.astype(jnp.int32)
    key = r * N + c
    valid = jnp.concatenate(
        [jnp.ones((1,), jnp.bool_), key[1:] > key[:-1]])
    vmask3 = valid.astype(jnp.float32).reshape(_E // _MLP_B, 1, _MLP_B)
    r_eff = jnp.where(valid, r, N)

    zeros = jnp.zeros((N + 8, _W), jnp.float32)

    We1a = jnp.pad(We1[:C], ((0, _W - C), (0, 0)))
    We1b = jnp.pad(We1[C:], ((0, _W - C), (0, 0)))
    be1r = be1.reshape(1, -1)
    be2r = be2.reshape(1, -1)
    We3p = jnp.pad(We3, ((0, 0), (0, 127)))
    be3r = jnp.broadcast_to(be3.reshape(1, 1), (1, 128))
    W2p = jnp.pad(W2, ((0, _W - C), (0, 0)))

    def lane_pad(a):
        a = a.astype(jnp.bfloat16).astype(jnp.float32)
        return jnp.pad(a, ((0, 8), (0, _W - C)))

    xW1 = _tc_matmul(x, W1)
    xW1p = lane_pad(xW1)
    a1p = _sc_spmm(xW1p, r_eff, c, None, zeros, N)
    q1 = _tc_combine_relu_matmul(a1p, W2p)
    embp = _sc_spmm(lane_pad(q1), r_eff, c, None, zeros, N)
    emb = _tc_combine(embp, _W)

    f1, f2 = _sc_gather2(emb, r, c)
    vb = _tc_edge_mlp(f1, f2, vmask3, We1a, We1b, be1r, We2, be2r,
                      We3p, be3r)
    vb = vb.astype(jnp.bfloat16).astype(jnp.float32)

    a2p = _sc_spmm(xW1p, r_eff, c, vb, zeros, N)
    q2 = _tc_combine_relu_matmul(a2p, W2p)
    zp = _sc_spmm(lane_pad(q2), r_eff, c, vb, zeros, N)
    return _tc_combine(zp, C)
